# baseline (speedup 1.0000x reference)
import functools

import jax
import jax.numpy as jnp
from jax.experimental import pallas as pl
from jax.experimental.pallas import tpu as pltpu

EMB = 32
NODE_DIM = 4
NEG = -1e20


def _fused_kernel(xa_ref, xb_ref, nfm_ref, reach_ref, act_ref, t1_ref, t2_ref,
                  t3_ref, t4_ref, t5_ref, t5b_ref,
                  out_nl_ref, out_ap_ref,
                  m_scr, base_scr, s_scr, musum_scr, *, n_nodes, tile, k_steps):
    k = pl.program_id(1)

    @pl.when(k == 0)
    def _init():
        nfm = nfm_ref[0]
        base = jax.lax.dot_general(
            nfm, t1_ref[...], (((1,), (0,)), ((), ())),
            preferred_element_type=jnp.float32)
        base_scr[...] = base
        mu1 = jnp.maximum(base, 0.0)
        m = jax.lax.dot_general(
            mu1, t2_ref[...], (((1,), (0,)), ((), ())),
            preferred_element_type=jnp.float32)
        zpad = jnp.zeros((NODE_DIM, EMB), jnp.float32)
        m_scr[...] = jnp.concatenate([zpad, m, zpad], axis=0)
        musum_scr[...] = jnp.zeros((1, EMB), jnp.float32)

    for i, x_ref in enumerate((xa_ref, xb_ref)):
        xt = x_ref[0]
        wm = jax.lax.dot_general(
            xt, m_scr[0:n_nodes + NODE_DIM + 1, :], (((1,), (0,)), ((), ())),
            preferred_element_type=jnp.float32)
        row0 = (2 * k + i) * tile
        base_t = base_scr[pl.ds(row0, tile), :]
        mu2 = jnp.maximum(base_t + wm, 0.0)
        musum_scr[...] += jnp.sum(mu2, axis=0, keepdims=True)
        loc = jnp.maximum(jax.lax.dot_general(
            mu2, t4_ref[...], (((1,), (0,)), ((), ())),
            preferred_element_type=jnp.float32), 0.0)
        s = jax.lax.dot_general(
            loc, t5_ref[EMB:2 * EMB, :], (((1,), (0,)), ((), ())),
            preferred_element_type=jnp.float32)
        s_scr[pl.ds(row0, tile), :] = s

    @pl.when(k == k_steps - 1)
    def _finish():
        g = jnp.maximum(jax.lax.dot_general(
            musum_scr[...], t3_ref[...], (((1,), (0,)), ((), ())),
            preferred_element_type=jnp.float32), 0.0)
        c = jax.lax.dot_general(
            g, t5_ref[0:EMB, :], (((1,), (0,)), ((), ())),
            preferred_element_type=jnp.float32)[0, 0] + t5b_ref[0, 0]
        logits = s_scr[...] + c
        reach = reach_ref[0]
        logits = jnp.where(reach != 0.0, logits, NEG)
        mx = jnp.max(logits)
        lse = mx + jnp.log(jnp.sum(jnp.exp(logits - mx)))
        norm = logits - lse
        out_nl_ref[0] = norm
        a = act_ref[0, 0, 0]
        idx = jax.lax.broadcasted_iota(jnp.int32, (n_nodes, 1), 0)
        out_ap_ref[0] = jnp.sum(jnp.where(idx == a, norm, 0.0),
                                axis=0, keepdims=True)


@jax.jit
def kernel(X, actions, theta1, theta2, theta3, theta4, theta5, theta5_b):
    if X.ndim == 2:
        X = X[None, ...]
    b_sz, n_nodes, row = X.shape
    tile = 512
    k_steps = n_nodes // (2 * tile)

    nfm = X[:, :, :NODE_DIM]
    reach = X[:, :, row - 1:row]
    acts = actions.astype(jnp.int32).reshape(b_sz, 1, 1)
    t5b = theta5_b.reshape(1, 1)

    grid = (b_sz, k_steps)
    kern = functools.partial(_fused_kernel, n_nodes=n_nodes, tile=tile,
                             k_steps=k_steps)
    norm_nl, act_p = pl.pallas_call(
        kern,
        grid=grid,
        in_specs=[
            pl.BlockSpec((1, tile, row), lambda b, k: (b, 2 * k, 0)),
            pl.BlockSpec((1, tile, row), lambda b, k: (b, 2 * k + 1, 0)),
            pl.BlockSpec((1, n_nodes, NODE_DIM), lambda b, k: (b, 0, 0)),
            pl.BlockSpec((1, n_nodes, 1), lambda b, k: (b, 0, 0)),
            pl.BlockSpec((1, 1, 1), lambda b, k: (b, 0, 0)),
            pl.BlockSpec((NODE_DIM, EMB), lambda b, k: (0, 0)),
            pl.BlockSpec((EMB, EMB), lambda b, k: (0, 0)),
            pl.BlockSpec((EMB, EMB), lambda b, k: (0, 0)),
            pl.BlockSpec((EMB, EMB), lambda b, k: (0, 0)),
            pl.BlockSpec((2 * EMB, 1), lambda b, k: (0, 0)),
            pl.BlockSpec((1, 1), lambda b, k: (0, 0)),
        ],
        out_specs=[
            pl.BlockSpec((1, n_nodes, 1), lambda b, k: (b, 0, 0)),
            pl.BlockSpec((1, 1, 1), lambda b, k: (b, 0, 0)),
        ],
        out_shape=[
            jax.ShapeDtypeStruct((b_sz, n_nodes, 1), jnp.float32),
            jax.ShapeDtypeStruct((b_sz, 1, 1), jnp.float32),
        ],
        scratch_shapes=[
            pltpu.VMEM((n_nodes + 2 * NODE_DIM, EMB), jnp.float32),
            pltpu.VMEM((n_nodes, EMB), jnp.float32),
            pltpu.VMEM((n_nodes, 1), jnp.float32),
            pltpu.VMEM((1, EMB), jnp.float32),
        ],
        compiler_params=pltpu.CompilerParams(
            dimension_semantics=("arbitrary", "arbitrary")),
    )(X, X, nfm, reach, acts, theta1, theta2, theta3, theta4, theta5, t5b)

    return norm_nl.reshape(b_sz, n_nodes), act_p.reshape(b_sz, 1)


# speedup vs baseline: 1.0000x; 1.0000x over previous
import functools

import jax
import jax.numpy as jnp
from jax.experimental import pallas as pl
from jax.experimental.pallas import tpu as pltpu

EMB = 32
NODE_DIM = 4
NEG = -1e20


def _fused_kernel(xa_ref, xb_ref, nfm_ref, reach_ref, act_ref, t1_ref, t2_ref,
                  t3_ref, t4_ref, t5_ref, t5b_ref,
                  out_nl_ref, out_ap_ref,
                  m_scr, base_scr, s_scr, musum_scr, *, n_nodes, tile, k_steps):
    k = pl.program_id(1)

    @pl.when(k == 0)
    def _init():
        nfm = nfm_ref[0]
        base = jax.lax.dot_general(
            nfm, t1_ref[...], (((1,), (0,)), ((), ())),
            preferred_element_type=jnp.float32)
        base_scr[...] = base
        mu1 = jnp.maximum(base, 0.0)
        m = jax.lax.dot_general(
            mu1, t2_ref[...], (((1,), (0,)), ((), ())),
            preferred_element_type=jnp.float32)
        zpad = jnp.zeros((NODE_DIM, EMB), jnp.float32)
        m_scr[...] = jnp.concatenate([zpad, m, zpad], axis=0).astype(jnp.bfloat16)
        musum_scr[...] = jnp.zeros((1, EMB), jnp.float32)

    for i, x_ref in enumerate((xa_ref, xb_ref)):
        xt = x_ref[0]
        wm = jax.lax.dot_general(
            xt.astype(jnp.bfloat16), m_scr[0:n_nodes + NODE_DIM + 1, :],
            (((1,), (0,)), ((), ())),
            preferred_element_type=jnp.float32)
        row0 = (2 * k + i) * tile
        base_t = base_scr[pl.ds(row0, tile), :]
        mu2 = jnp.maximum(base_t + wm, 0.0)
        musum_scr[...] += jnp.sum(mu2, axis=0, keepdims=True)
        loc = jnp.maximum(jax.lax.dot_general(
            mu2, t4_ref[...], (((1,), (0,)), ((), ())),
            preferred_element_type=jnp.float32), 0.0)
        s = jax.lax.dot_general(
            loc, t5_ref[EMB:2 * EMB, :], (((1,), (0,)), ((), ())),
            preferred_element_type=jnp.float32)
        s_scr[pl.ds(row0, tile), :] = s

    @pl.when(k == k_steps - 1)
    def _finish():
        g = jnp.maximum(jax.lax.dot_general(
            musum_scr[...], t3_ref[...], (((1,), (0,)), ((), ())),
            preferred_element_type=jnp.float32), 0.0)
        c = jax.lax.dot_general(
            g, t5_ref[0:EMB, :], (((1,), (0,)), ((), ())),
            preferred_element_type=jnp.float32)[0, 0] + t5b_ref[0, 0]
        logits = s_scr[...] + c
        reach = reach_ref[0]
        logits = jnp.where(reach != 0.0, logits, NEG)
        mx = jnp.max(logits)
        lse = mx + jnp.log(jnp.sum(jnp.exp(logits - mx)))
        norm = logits - lse
        out_nl_ref[0] = norm
        a = act_ref[0, 0, 0]
        idx = jax.lax.broadcasted_iota(jnp.int32, (n_nodes, 1), 0)
        out_ap_ref[0] = jnp.sum(jnp.where(idx == a, norm, 0.0),
                                axis=0, keepdims=True)


@jax.jit
def kernel(X, actions, theta1, theta2, theta3, theta4, theta5, theta5_b):
    if X.ndim == 2:
        X = X[None, ...]
    b_sz, n_nodes, row = X.shape
    tile = 512
    k_steps = n_nodes // (2 * tile)

    nfm = X[:, :, :NODE_DIM]
    reach = X[:, :, row - 1:row]
    acts = actions.astype(jnp.int32).reshape(b_sz, 1, 1)
    t5b = theta5_b.reshape(1, 1)

    grid = (b_sz, k_steps)
    kern = functools.partial(_fused_kernel, n_nodes=n_nodes, tile=tile,
                             k_steps=k_steps)
    norm_nl, act_p = pl.pallas_call(
        kern,
        grid=grid,
        in_specs=[
            pl.BlockSpec((1, tile, row), lambda b, k: (b, 2 * k, 0)),
            pl.BlockSpec((1, tile, row), lambda b, k: (b, 2 * k + 1, 0)),
            pl.BlockSpec((1, n_nodes, NODE_DIM), lambda b, k: (b, 0, 0)),
            pl.BlockSpec((1, n_nodes, 1), lambda b, k: (b, 0, 0)),
            pl.BlockSpec((1, 1, 1), lambda b, k: (b, 0, 0)),
            pl.BlockSpec((NODE_DIM, EMB), lambda b, k: (0, 0)),
            pl.BlockSpec((EMB, EMB), lambda b, k: (0, 0)),
            pl.BlockSpec((EMB, EMB), lambda b, k: (0, 0)),
            pl.BlockSpec((EMB, EMB), lambda b, k: (0, 0)),
            pl.BlockSpec((2 * EMB, 1), lambda b, k: (0, 0)),
            pl.BlockSpec((1, 1), lambda b, k: (0, 0)),
        ],
        out_specs=[
            pl.BlockSpec((1, n_nodes, 1), lambda b, k: (b, 0, 0)),
            pl.BlockSpec((1, 1, 1), lambda b, k: (b, 0, 0)),
        ],
        out_shape=[
            jax.ShapeDtypeStruct((b_sz, n_nodes, 1), jnp.float32),
            jax.ShapeDtypeStruct((b_sz, 1, 1), jnp.float32),
        ],
        scratch_shapes=[
            pltpu.VMEM((n_nodes + 2 * NODE_DIM, EMB), jnp.bfloat16),
            pltpu.VMEM((n_nodes, EMB), jnp.float32),
            pltpu.VMEM((n_nodes, 1), jnp.float32),
            pltpu.VMEM((1, EMB), jnp.float32),
        ],
        compiler_params=pltpu.CompilerParams(
            dimension_semantics=("arbitrary", "arbitrary")),
    )(X, X, nfm, reach, acts, theta1, theta2, theta3, theta4, theta5, t5b)

    return norm_nl.reshape(b_sz, n_nodes), act_p.reshape(b_sz, 1)


# speedup vs baseline: 1.1040x; 1.1040x over previous
import jax
import jax.numpy as jnp
from jax.experimental import pallas as pl

def _probe(x_ref, nfm_ref, reach_ref, o_ref, onl_ref):
    b = pl.program_id(0); k = pl.program_id(1)
    @pl.when((b == 0) & (k == 0))
    def _():
        o_ref[...] = jnp.zeros_like(o_ref)
    o_ref[...] += jnp.sum(x_ref[...], axis=(0, 1), keepdims=True)[0]
    @pl.when(k == 3)
    def _():
        onl_ref[0] = jnp.sum(nfm_ref[0], axis=1, keepdims=True) + reach_ref[0]


@jax.jit
def kernel(X, actions, theta1, theta2, theta3, theta4, theta5, theta5_b):
    b_sz, n, row = X.shape
    tile = 512
    nfm = X[:, :, :4]
    reach = X[:, :, row - 1:row]
    out, onl = pl.pallas_call(
        _probe,
        grid=(b_sz, n // tile),
        in_specs=[pl.BlockSpec((1, tile, row), lambda b, k: (b, k, 0)),
                  pl.BlockSpec((1, n, 4), lambda b, k: (b, 0, 0)),
                  pl.BlockSpec((1, n, 1), lambda b, k: (b, 0, 0))],
        out_specs=[pl.BlockSpec((1, row), lambda b, k: (0, 0)),
                   pl.BlockSpec((1, n, 1), lambda b, k: (b, 0, 0))],
        out_shape=[jax.ShapeDtypeStruct((1, row), jnp.float32),
                   jax.ShapeDtypeStruct((b_sz, n, 1), jnp.float32)],
    )(X, nfm, reach)
    nl = jnp.zeros((b_sz, n), jnp.float32) + out[0, 0] + onl[..., 0]
    return nl, jnp.zeros((b_sz, 1), jnp.float32)


# speedup vs baseline: 1.1338x; 1.0270x over previous
import functools

import jax
import jax.numpy as jnp
from jax.experimental import pallas as pl
from jax.experimental.pallas import tpu as pltpu

EMB = 32
NODE_DIM = 4
NEG = -1e20


def _fused_kernel(x_ref, nfmt_ref, reach_ref, act_ref, t1_ref, t2_ref,
                  t3_ref, t4_ref, t5_ref, t5b_ref,
                  out_nl_ref, out_ap_ref,
                  m_scr, base_scr, s_scr, musum_scr, *, n_nodes, tile, k_steps):
    k = pl.program_id(1)

    @pl.when(k == 0)
    def _init():
        nfm_t = nfmt_ref[0]
        base = jax.lax.dot_general(
            nfm_t, t1_ref[...], (((0,), (0,)), ((), ())),
            preferred_element_type=jnp.float32)
        base_scr[...] = base
        mu1 = jnp.maximum(base, 0.0)
        m = jax.lax.dot_general(
            mu1, t2_ref[...], (((1,), (0,)), ((), ())),
            preferred_element_type=jnp.float32)
        zpad = jnp.zeros((NODE_DIM, EMB), jnp.float32)
        m_scr[...] = jnp.concatenate([zpad, m, zpad],
                                     axis=0).astype(jnp.bfloat16)
        musum_scr[...] = jnp.zeros((1, EMB), jnp.float32)

    xt = x_ref[0]
    wm = jax.lax.dot_general(
        xt.astype(jnp.bfloat16), m_scr[0:n_nodes + NODE_DIM + 1, :],
        (((1,), (0,)), ((), ())),
        preferred_element_type=jnp.float32)
    base_t = base_scr[pl.ds(k * tile, tile), :]
    mu2 = jnp.maximum(base_t + wm, 0.0)
    musum_scr[...] += jnp.sum(mu2, axis=0, keepdims=True)
    loc = jnp.maximum(jax.lax.dot_general(
        mu2, t4_ref[...], (((1,), (0,)), ((), ())),
        preferred_element_type=jnp.float32), 0.0)
    s_row = jax.lax.dot_general(
        t5_ref[EMB:2 * EMB, 0:1], loc, (((0,), (1,)), ((), ())),
        preferred_element_type=jnp.float32)
    s_scr[:, pl.ds(k * tile, tile)] = s_row

    @pl.when(k == k_steps - 1)
    def _finish():
        g = jnp.maximum(jax.lax.dot_general(
            musum_scr[...], t3_ref[...], (((1,), (0,)), ((), ())),
            preferred_element_type=jnp.float32), 0.0)
        c = jax.lax.dot_general(
            g, t5_ref[0:EMB, 0:1], (((1,), (0,)), ((), ())),
            preferred_element_type=jnp.float32)[0, 0] + t5b_ref[0, 0]
        logits = s_scr[...] + c
        reach = reach_ref[0]
        logits = jnp.where(reach != 0.0, logits, NEG)
        mx = jnp.max(logits)
        lse = mx + jnp.log(jnp.sum(jnp.exp(logits - mx)))
        norm = logits - lse
        out_nl_ref[0] = norm
        a = act_ref[0, 0, 0]
        idx = jax.lax.broadcasted_iota(jnp.int32, (1, n_nodes), 1)
        out_ap_ref[0] = jnp.sum(jnp.where(idx == a, norm, 0.0),
                                axis=1, keepdims=True)


@jax.jit
def kernel(X, actions, theta1, theta2, theta3, theta4, theta5, theta5_b):
    if X.ndim == 2:
        X = X[None, ...]
    b_sz, n_nodes, row = X.shape
    tile = 512
    k_steps = n_nodes // tile

    nfm_t = jnp.swapaxes(X[:, :, :NODE_DIM], 1, 2)
    reach = X[:, :, row - 1].reshape(b_sz, 1, n_nodes)
    acts = actions.astype(jnp.int32).reshape(b_sz, 1, 1)
    t5p = jnp.pad(theta5, ((0, 0), (0, 127)))
    t5b = theta5_b.reshape(1, 1)

    grid = (b_sz, k_steps)
    kern = functools.partial(_fused_kernel, n_nodes=n_nodes, tile=tile,
                             k_steps=k_steps)
    norm_nl, act_p = pl.pallas_call(
        kern,
        grid=grid,
        in_specs=[
            pl.BlockSpec((1, tile, row), lambda b, k: (b, k, 0)),
            pl.BlockSpec((1, NODE_DIM, n_nodes), lambda b, k: (b, 0, 0)),
            pl.BlockSpec((1, 1, n_nodes), lambda b, k: (b, 0, 0)),
            pl.BlockSpec((1, 1, 1), lambda b, k: (b, 0, 0)),
            pl.BlockSpec((NODE_DIM, EMB), lambda b, k: (0, 0)),
            pl.BlockSpec((EMB, EMB), lambda b, k: (0, 0)),
            pl.BlockSpec((EMB, EMB), lambda b, k: (0, 0)),
            pl.BlockSpec((EMB, EMB), lambda b, k: (0, 0)),
            pl.BlockSpec((2 * EMB, 128), lambda b, k: (0, 0)),
            pl.BlockSpec((1, 1), lambda b, k: (0, 0)),
        ],
        out_specs=[
            pl.BlockSpec((1, 1, n_nodes), lambda b, k: (b, 0, 0)),
            pl.BlockSpec((1, 1, 1), lambda b, k: (b, 0, 0)),
        ],
        out_shape=[
            jax.ShapeDtypeStruct((b_sz, 1, n_nodes), jnp.float32),
            jax.ShapeDtypeStruct((b_sz, 1, 1), jnp.float32),
        ],
        scratch_shapes=[
            pltpu.VMEM((n_nodes + 2 * NODE_DIM, EMB), jnp.bfloat16),
            pltpu.VMEM((n_nodes, EMB), jnp.float32),
            pltpu.VMEM((1, n_nodes), jnp.float32),
            pltpu.VMEM((1, EMB), jnp.float32),
        ],
        compiler_params=pltpu.CompilerParams(
            dimension_semantics=("arbitrary", "arbitrary")),
    )(X, nfm_t, reach, acts, theta1, theta2, theta3, theta4, t5p, t5b)

    return norm_nl.reshape(b_sz, n_nodes), act_p.reshape(b_sz, 1)
